# Initial kernel scaffold; baseline (speedup 1.0000x reference)
import jax
import jax.numpy as jnp
from jax.experimental import pallas as pl


def kernel(point_coord, labels, pc_range, label_weight):
    raise NotImplementedError("write your pallas kernel here")


# speedup vs baseline: 3.7360x; 3.7360x over previous
import functools
import math

import jax
import jax.numpy as jnp
import numpy as np
from jax import lax
from jax.experimental import pallas as pl
from jax.experimental.pallas import tpu as pltpu
from jax.experimental.pallas import tpu_sc as plsc

_NUM_FEATS = 128
_TEMPERATURE = 10000.0
_NC = 2
_NS = 16
_NW = _NC * _NS
_CH = 128


def _sc_gather(table, idx3):
    nw, n_ch, ch = idx3.shape
    d = table.shape[1]
    b = nw * n_ch * ch
    mesh = plsc.VectorSubcoreMesh(core_axis_name="c", subcore_axis_name="s")

    @functools.partial(
        pl.kernel,
        mesh=mesh,
        out_type=jax.ShapeDtypeStruct((b, d), jnp.float32),
        scratch_types=[
            pltpu.VMEM((n_ch, ch), jnp.int32),
            pltpu.VMEM((ch, d), jnp.float32),
            pltpu.VMEM((ch, d), jnp.float32),
            pltpu.SemaphoreType.DMA,
            pltpu.SemaphoreType.DMA,
        ],
    )
    def gather_kernel(idx_hbm, table_hbm, out_hbm, idx_v, rows0, rows1, sem0, sem1):
        wid = lax.axis_index("s") * _NC + lax.axis_index("c")
        base = wid * (n_ch * ch)
        pltpu.sync_copy(idx_hbm.at[wid], idx_v)
        bufs = (rows0, rows1)
        sems = (sem0, sem1)
        copies = [pltpu.async_copy(table_hbm.at[idx_v.at[0]], bufs[0], sems[0])]
        for c in range(n_ch):
            if c + 1 < n_ch:
                copies.append(
                    pltpu.async_copy(
                        table_hbm.at[idx_v.at[c + 1]], bufs[(c + 1) % 2], sems[(c + 1) % 2]
                    )
                )
            copies[c].wait()
            pltpu.sync_copy(bufs[c % 2], out_hbm.at[pl.ds(base + c * ch, ch)])

    return gather_kernel(idx3, table)


_TWO_OVER_PI = float(np.float32(2.0 / np.pi))
_MAGIC = 12582912.0
_P1 = float(np.float32(np.pi / 2))
_P2 = float(np.float32(np.pi / 2 - np.float64(np.float32(np.pi / 2))))
_P3 = float(
    np.float32(np.pi / 2 - np.float64(np.float32(np.pi / 2)) - np.float64(_P2))
)


def _sincos_sel(t, parity):
    j = (t * _TWO_OVER_PI + _MAGIC) - _MAGIC
    r = t - j * _P1
    r = r - j * _P2
    q = j.astype(jnp.int32) + parity
    r2 = r * r
    s = r + r * r2 * (-1.6658333e-1 + r2 * 8.0187609e-3)
    c = 1.0 + r2 * (-4.9985713e-1 + r2 * 4.0905953e-2)
    v = jnp.where((q & 1) == 0, s, c)
    return jnp.where((q & 2) == 0, v, -v)


def _tc_combine(gathered, coord, pc_range, inv_dim):
    n, d = gathered.shape
    nb = 512

    def body(pc_ref, invd_ref, coord_ref, g_ref, out_ref):
        scale = 2.0 * math.pi
        ax = scale / (pc_ref[3] - pc_ref[0])
        ay = scale / (pc_ref[4] - pc_ref[1])
        invd = invd_ref[0:1, :]
        x = coord_ref[:, 0:1]
        y = coord_ref[:, 1:2]
        tx = ((x - pc_ref[0]) * ax) * invd
        ty = ((y - pc_ref[1]) * ay) * invd
        ii = lax.broadcasted_iota(jnp.int32, (nb, _NUM_FEATS), 1)
        parity = ii & 1
        pos_x = _sincos_sel(tx, parity)
        pos_y = _sincos_sel(ty, parity)
        out_ref[:, 0:256] = jnp.zeros((nb, 256), jnp.float32)
        out_ref[:, 256:384] = g_ref[:, 0:128] + pos_x
        out_ref[:, 384:512] = g_ref[:, 128:256] + pos_y

    return pl.pallas_call(
        body,
        grid=(n // nb,),
        in_specs=[
            pl.BlockSpec(memory_space=pltpu.SMEM),
            pl.BlockSpec((1, _NUM_FEATS), lambda i: (0, 0)),
            pl.BlockSpec((nb, 3), lambda i: (i, 0)),
            pl.BlockSpec((nb, d), lambda i: (i, 0)),
        ],
        out_specs=pl.BlockSpec((nb, 2 * d), lambda i: (i, 0)),
        out_shape=jax.ShapeDtypeStruct((n, 2 * d), jnp.float32),
    )(pc_range, inv_dim, coord, gathered)


def kernel(point_coord, labels, pc_range, label_weight):
    n = labels.shape[0]
    idx3 = labels.reshape(_NW, n // (_NW * _CH), _CH)
    gathered = _sc_gather(label_weight, idx3)
    i = np.arange(_NUM_FEATS, dtype=np.float32)
    dim_t = (_TEMPERATURE ** (2.0 * np.floor(i / 2.0) / _NUM_FEATS)).astype(np.float32)
    inv_dim = jnp.asarray((1.0 / dim_t).reshape(1, _NUM_FEATS))
    out = _tc_combine(gathered, point_coord[0], pc_range, inv_dim)
    return out[None]


# speedup vs baseline: 3.7894x; 1.0143x over previous
import functools
import math

import jax
import jax.numpy as jnp
import numpy as np
from jax import lax
from jax.experimental import pallas as pl
from jax.experimental.pallas import tpu as pltpu
from jax.experimental.pallas import tpu_sc as plsc

_NUM_FEATS = 128
_TEMPERATURE = 10000.0
_NC = 2
_NS = 16
_NW = _NC * _NS
_CH = 128


def _sc_gather(table, labels):
    b = labels.shape[0]
    d = table.shape[1]
    per_w = b // _NW
    n_ch = per_w // _CH
    nbuf = 3
    mesh = plsc.VectorSubcoreMesh(core_axis_name="c", subcore_axis_name="s")

    @functools.partial(
        pl.kernel,
        mesh=mesh,
        out_type=jax.ShapeDtypeStruct((b, d), jnp.float32),
        scratch_types=[
            pltpu.VMEM((n_ch, _CH), jnp.int32),
            pltpu.VMEM((_CH, d), jnp.float32),
            pltpu.VMEM((_CH, d), jnp.float32),
            pltpu.VMEM((_CH, d), jnp.float32),
            pltpu.SemaphoreType.DMA,
            pltpu.SemaphoreType.DMA,
            pltpu.SemaphoreType.DMA,
            pltpu.SemaphoreType.DMA,
            pltpu.SemaphoreType.DMA,
            pltpu.SemaphoreType.DMA,
            pltpu.SemaphoreType.DMA,
        ],
    )
    def gather_kernel(lab_hbm, table_hbm, out_hbm, idx_v, r0, r1, r2,
                      isem, g0, g1, g2, s0, s1, s2):
        wid = lax.axis_index("s") * _NC + lax.axis_index("c")
        base = wid * per_w
        bufs = (r0, r1, r2)
        gsems = (g0, g1, g2)
        ssems = (s0, s1, s2)
        idx_copies = [
            pltpu.async_copy(lab_hbm.at[pl.ds(base + c * _CH, _CH)], idx_v.at[c], isem)
            for c in range(n_ch)
        ]
        for cp in idx_copies:
            cp.wait()
        gathers = {}
        stores = {}
        for c in range(min(nbuf, n_ch)):
            gathers[c] = pltpu.async_copy(
                table_hbm.at[idx_v.at[c]], bufs[c % nbuf], gsems[c % nbuf]
            )
        for c in range(n_ch):
            gathers[c].wait()
            stores[c] = pltpu.async_copy(
                bufs[c % nbuf], out_hbm.at[pl.ds(base + c * _CH, _CH)], ssems[c % nbuf]
            )
            nxt = c + nbuf
            if nxt < n_ch:
                stores[c].wait()
                gathers[nxt] = pltpu.async_copy(
                    table_hbm.at[idx_v.at[nxt]], bufs[nxt % nbuf], gsems[nxt % nbuf]
                )
        for c in range(max(0, n_ch - nbuf), n_ch):
            stores[c].wait()

    return gather_kernel(labels, table)


_TWO_OVER_PI = float(np.float32(2.0 / np.pi))
_MAGIC = 12582912.0
_P1 = float(np.float32(np.pi / 2))
_P2 = float(np.float32(np.pi / 2 - np.float64(np.float32(np.pi / 2))))
_P3 = float(
    np.float32(np.pi / 2 - np.float64(np.float32(np.pi / 2)) - np.float64(_P2))
)


def _sincos_sel(t, parity):
    j = (t * _TWO_OVER_PI + _MAGIC) - _MAGIC
    r = t - j * _P1
    r = r - j * _P2
    q = j.astype(jnp.int32) + parity
    r2 = r * r
    s = r + r * r2 * (-1.6658333e-1 + r2 * 8.0187609e-3)
    c = 1.0 + r2 * (-4.9985713e-1 + r2 * 4.0905953e-2)
    v = jnp.where((q & 1) == 0, s, c)
    return jnp.where((q & 2) == 0, v, -v)


def _tc_combine(gathered, coord, pc_range, inv_dim):
    n, d = gathered.shape
    nb = 512

    def body(pc_ref, invd_ref, coord_ref, g_ref, out_ref):
        scale = 2.0 * math.pi
        ax = scale / (pc_ref[3] - pc_ref[0])
        ay = scale / (pc_ref[4] - pc_ref[1])
        invd = invd_ref[0:1, :]
        x = coord_ref[:, 0:1]
        y = coord_ref[:, 1:2]
        tx = ((x - pc_ref[0]) * ax) * invd
        ty = ((y - pc_ref[1]) * ay) * invd
        ii = lax.broadcasted_iota(jnp.int32, (nb, _NUM_FEATS), 1)
        parity = ii & 1
        pos_x = _sincos_sel(tx, parity)
        pos_y = _sincos_sel(ty, parity)
        out_ref[:, 0:256] = jnp.zeros((nb, 256), jnp.float32)
        out_ref[:, 256:384] = g_ref[:, 0:128] + pos_x
        out_ref[:, 384:512] = g_ref[:, 128:256] + pos_y

    return pl.pallas_call(
        body,
        grid=(n // nb,),
        in_specs=[
            pl.BlockSpec(memory_space=pltpu.SMEM),
            pl.BlockSpec((1, _NUM_FEATS), lambda i: (0, 0)),
            pl.BlockSpec((nb, 3), lambda i: (i, 0)),
            pl.BlockSpec((nb, d), lambda i: (i, 0)),
        ],
        out_specs=pl.BlockSpec((nb, 2 * d), lambda i: (i, 0)),
        out_shape=jax.ShapeDtypeStruct((n, 2 * d), jnp.float32),
    )(pc_range, inv_dim, coord, gathered)


def kernel(point_coord, labels, pc_range, label_weight):
    gathered = _sc_gather(label_weight, labels)
    i = np.arange(_NUM_FEATS, dtype=np.float32)
    dim_t = (_TEMPERATURE ** (2.0 * np.floor(i / 2.0) / _NUM_FEATS)).astype(np.float32)
    inv_dim = jnp.asarray((1.0 / dim_t).reshape(1, _NUM_FEATS))
    out = _tc_combine(gathered, point_coord[0], pc_range, inv_dim)
    return out[None]


# speedup vs baseline: 3.9134x; 1.0327x over previous
import functools
import math

import jax
import jax.numpy as jnp
import numpy as np
from jax import lax
from jax.experimental import pallas as pl
from jax.experimental.pallas import tpu as pltpu
from jax.experimental.pallas import tpu_sc as plsc

_NUM_FEATS = 128
_TEMPERATURE = 10000.0
_NC = 2
_NS = 16
_NW = _NC * _NS
_CH = 128


def _sc_gather(table, labels):
    b = labels.shape[0]
    d = table.shape[1]
    per_w = b // _NW
    n_ch = per_w // _CH
    nbuf = 3
    mesh = plsc.VectorSubcoreMesh(core_axis_name="c", subcore_axis_name="s")

    @functools.partial(
        pl.kernel,
        mesh=mesh,
        out_type=jax.ShapeDtypeStruct((b, d), jnp.float32),
        scratch_types=[
            pltpu.VMEM((n_ch, _CH), jnp.int32),
            pltpu.VMEM((_CH, d), jnp.float32),
            pltpu.VMEM((_CH, d), jnp.float32),
            pltpu.VMEM((_CH, d), jnp.float32),
            pltpu.SemaphoreType.DMA,
            pltpu.SemaphoreType.DMA,
            pltpu.SemaphoreType.DMA,
            pltpu.SemaphoreType.DMA,
            pltpu.SemaphoreType.DMA,
            pltpu.SemaphoreType.DMA,
            pltpu.SemaphoreType.DMA,
        ],
    )
    def gather_kernel(lab_hbm, table_hbm, out_hbm, idx_v, r0, r1, r2,
                      isem, g0, g1, g2, s0, s1, s2):
        wid = lax.axis_index("s") * _NC + lax.axis_index("c")
        base = wid * per_w
        bufs = (r0, r1, r2)
        gsems = (g0, g1, g2)
        ssems = (s0, s1, s2)
        idx_copies = [
            pltpu.async_copy(lab_hbm.at[pl.ds(base + c * _CH, _CH)], idx_v.at[c], isem)
            for c in range(n_ch)
        ]
        for cp in idx_copies:
            cp.wait()
        gathers = {}
        stores = {}
        for c in range(min(nbuf, n_ch)):
            gathers[c] = pltpu.async_copy(
                table_hbm.at[idx_v.at[c]], bufs[c % nbuf], gsems[c % nbuf]
            )
        for c in range(n_ch):
            gathers[c].wait()
            stores[c] = pltpu.async_copy(
                bufs[c % nbuf], out_hbm.at[pl.ds(base + c * _CH, _CH)], ssems[c % nbuf]
            )
            nxt = c + nbuf
            if nxt < n_ch:
                stores[c].wait()
                gathers[nxt] = pltpu.async_copy(
                    table_hbm.at[idx_v.at[nxt]], bufs[nxt % nbuf], gsems[nxt % nbuf]
                )
        for c in range(max(0, n_ch - nbuf), n_ch):
            stores[c].wait()

    return gather_kernel(labels, table)


_TWO_OVER_PI = float(np.float32(2.0 / np.pi))
_MAGIC = 12582912.0
_P1 = float(np.float32(np.pi / 2))
_P2 = float(np.float32(np.pi / 2 - np.float64(np.float32(np.pi / 2))))
_P3 = float(
    np.float32(np.pi / 2 - np.float64(np.float32(np.pi / 2)) - np.float64(_P2))
)


def _sincos_sel(t, parity):
    j = (t * _TWO_OVER_PI + _MAGIC) - _MAGIC
    r = t - j * _P1
    r = r - j * _P2
    q = j.astype(jnp.int32) + parity
    r2 = r * r
    s = r + r * r2 * (-1.6658333e-1 + r2 * 8.0187609e-3)
    c = 1.0 + r2 * (-4.9985713e-1 + r2 * 4.0905953e-2)
    v = jnp.where((q & 1) == 0, s, c)
    return jnp.where((q & 2) == 0, v, -v)


def _tc_combine(gathered, coord_t, w, b):
    n, d = gathered.shape
    nb = 512

    def body(w_ref, b_ref, ct_ref, g_ref, out_ref):
        t = (
            lax.dot_general(
                ct_ref[...], w_ref[...], (((0,), (0,)), ((), ())),
                preferred_element_type=jnp.float32,
            )
            + b_ref[...]
        )
        ii = lax.broadcasted_iota(jnp.int32, (nb, 2 * _NUM_FEATS), 1)
        pos = _sincos_sel(t, ii & 1)
        out_ref[:, 0:256] = jnp.zeros((nb, 256), jnp.float32)
        out_ref[:, 256:512] = g_ref[...] + pos

    return pl.pallas_call(
        body,
        grid=(n // nb,),
        in_specs=[
            pl.BlockSpec((3, 2 * _NUM_FEATS), lambda i: (0, 0)),
            pl.BlockSpec((1, 2 * _NUM_FEATS), lambda i: (0, 0)),
            pl.BlockSpec((3, nb), lambda i: (0, i)),
            pl.BlockSpec((nb, d), lambda i: (i, 0)),
        ],
        out_specs=pl.BlockSpec((nb, 2 * d), lambda i: (i, 0)),
        out_shape=jax.ShapeDtypeStruct((n, 2 * d), jnp.float32),
    )(w, b, coord_t, gathered)


def kernel(point_coord, labels, pc_range, label_weight):
    gathered = _sc_gather(label_weight, labels)
    i = np.arange(_NUM_FEATS, dtype=np.float32)
    dim_t = (_TEMPERATURE ** (2.0 * np.floor(i / 2.0) / _NUM_FEATS)).astype(np.float32)
    invd = jnp.asarray(1.0 / dim_t)
    scale = 2.0 * math.pi
    ax = scale / (pc_range[3] - pc_range[0])
    ay = scale / (pc_range[4] - pc_range[1])
    zero = jnp.zeros((_NUM_FEATS,), jnp.float32)
    w = jnp.stack(
        [
            jnp.concatenate([ax * invd, zero]),
            jnp.concatenate([zero, ay * invd]),
            jnp.zeros((2 * _NUM_FEATS,), jnp.float32),
        ]
    )
    b = jnp.concatenate([(-pc_range[0] * ax) * invd, (-pc_range[1] * ay) * invd])[None, :]
    coord_t = point_coord[0].T
    out = _tc_combine(gathered, coord_t, w, b)
    return out[None]


# speedup vs baseline: 4.0075x; 1.0240x over previous
import functools
import math

import jax
import jax.numpy as jnp
import numpy as np
from jax import lax
from jax.experimental import pallas as pl
from jax.experimental.pallas import tpu as pltpu
from jax.experimental.pallas import tpu_sc as plsc

_NUM_FEATS = 128
_TEMPERATURE = 10000.0
_NC = 2
_NS = 16
_NW = _NC * _NS
_CH = 128


def _sc_gather(table, labels):
    b = labels.shape[0]
    d = table.shape[1]
    per_w = b // _NW
    n_ch = per_w // _CH
    nbuf = 3
    mesh = plsc.VectorSubcoreMesh(core_axis_name="c", subcore_axis_name="s")

    @functools.partial(
        pl.kernel,
        mesh=mesh,
        out_type=jax.ShapeDtypeStruct((b, d), jnp.float32),
        scratch_types=[
            pltpu.VMEM((n_ch, _CH), jnp.int32),
            pltpu.VMEM((_CH, d), jnp.float32),
            pltpu.VMEM((_CH, d), jnp.float32),
            pltpu.VMEM((_CH, d), jnp.float32),
            pltpu.SemaphoreType.DMA,
            pltpu.SemaphoreType.DMA,
            pltpu.SemaphoreType.DMA,
            pltpu.SemaphoreType.DMA,
            pltpu.SemaphoreType.DMA,
            pltpu.SemaphoreType.DMA,
            pltpu.SemaphoreType.DMA,
        ],
    )
    def gather_kernel(lab_hbm, table_hbm, out_hbm, idx_v, r0, r1, r2,
                      isem, g0, g1, g2, s0, s1, s2):
        wid = lax.axis_index("s") * _NC + lax.axis_index("c")
        base = wid * per_w
        bufs = (r0, r1, r2)
        gsems = (g0, g1, g2)
        ssems = (s0, s1, s2)
        idx_copies = [
            pltpu.async_copy(lab_hbm.at[pl.ds(base + c * _CH, _CH)], idx_v.at[c], isem)
            for c in range(n_ch)
        ]
        for cp in idx_copies:
            cp.wait()
        gathers = {}
        stores = {}
        for c in range(min(nbuf, n_ch)):
            gathers[c] = pltpu.async_copy(
                table_hbm.at[idx_v.at[c]], bufs[c % nbuf], gsems[c % nbuf]
            )
        for c in range(n_ch):
            gathers[c].wait()
            stores[c] = pltpu.async_copy(
                bufs[c % nbuf], out_hbm.at[pl.ds(base + c * _CH, _CH)], ssems[c % nbuf]
            )
            nxt = c + nbuf
            if nxt < n_ch:
                stores[c].wait()
                gathers[nxt] = pltpu.async_copy(
                    table_hbm.at[idx_v.at[nxt]], bufs[nxt % nbuf], gsems[nxt % nbuf]
                )
        for c in range(max(0, n_ch - nbuf), n_ch):
            stores[c].wait()

    return gather_kernel(labels, table)


_TWO_OVER_PI = float(np.float32(2.0 / np.pi))
_MAGIC = 12582912.0
_P1 = float(np.float32(np.pi / 2))
_P2 = float(np.float32(np.pi / 2 - np.float64(np.float32(np.pi / 2))))
_P3 = float(
    np.float32(np.pi / 2 - np.float64(np.float32(np.pi / 2)) - np.float64(_P2))
)


def _sincos_sel(t, parity):
    j = (t * _TWO_OVER_PI + _MAGIC) - _MAGIC
    r = t - j * _P1
    r = r - j * _P2
    q = j.astype(jnp.int32) + parity
    r2 = r * r
    s = r + r * r2 * (-1.6658333e-1 + r2 * 8.0187609e-3)
    c = 1.0 + r2 * (-4.9985713e-1 + r2 * 4.0905953e-2)
    v = jnp.where((q & 1) == 0, s, c)
    return jnp.where((q & 2) == 0, v, -v)


def _tc_combine(gathered, coord_t, w, b, out_rows, blk_off, prev=None):
    n, d = gathered.shape
    nb = 512

    def body(w_ref, b_ref, ct_ref, g_ref, *refs):
        out_ref = refs[-1]
        t = (
            lax.dot_general(
                ct_ref[...], w_ref[...], (((0,), (0,)), ((), ())),
                preferred_element_type=jnp.float32,
            )
            + b_ref[...]
        )
        ii = lax.broadcasted_iota(jnp.int32, (nb, 2 * _NUM_FEATS), 1)
        pos = _sincos_sel(t, ii & 1)
        out_ref[:, 0:256] = jnp.zeros((nb, 256), jnp.float32)
        out_ref[:, 256:512] = g_ref[...] + pos

    in_specs = [
        pl.BlockSpec((3, 2 * _NUM_FEATS), lambda i: (0, 0)),
        pl.BlockSpec((1, 2 * _NUM_FEATS), lambda i: (0, 0)),
        pl.BlockSpec((3, nb), lambda i: (0, i)),
        pl.BlockSpec((nb, d), lambda i: (i, 0)),
    ]
    inputs = [w, b, coord_t, gathered]
    io_alias = {}
    if prev is not None:
        in_specs.append(pl.BlockSpec(memory_space=pl.ANY))
        inputs.append(prev)
        io_alias = {4: 0}
    return pl.pallas_call(
        body,
        grid=(n // nb,),
        in_specs=in_specs,
        out_specs=pl.BlockSpec((nb, 2 * d), lambda i: (i + blk_off, 0)),
        out_shape=jax.ShapeDtypeStruct((out_rows, 2 * d), jnp.float32),
        input_output_aliases=io_alias,
    )(*inputs)


def kernel(point_coord, labels, pc_range, label_weight):
    n = labels.shape[0]
    half = n // 2
    g0 = _sc_gather(label_weight, labels[:half])
    g1 = _sc_gather(label_weight, labels[half:])
    i = np.arange(_NUM_FEATS, dtype=np.float32)
    dim_t = (_TEMPERATURE ** (2.0 * np.floor(i / 2.0) / _NUM_FEATS)).astype(np.float32)
    invd = jnp.asarray(1.0 / dim_t)
    scale = 2.0 * math.pi
    ax = scale / (pc_range[3] - pc_range[0])
    ay = scale / (pc_range[4] - pc_range[1])
    zero = jnp.zeros((_NUM_FEATS,), jnp.float32)
    w = jnp.stack(
        [
            jnp.concatenate([ax * invd, zero]),
            jnp.concatenate([zero, ay * invd]),
            jnp.zeros((2 * _NUM_FEATS,), jnp.float32),
        ]
    )
    b = jnp.concatenate([(-pc_range[0] * ax) * invd, (-pc_range[1] * ay) * invd])[None, :]
    coord_t = point_coord[0].T
    out0 = _tc_combine(g0, coord_t[:, :half], w, b, n, 0)
    out = _tc_combine(g1, coord_t[:, half:], w, b, n, half // 512, prev=out0)
    return out[None]


# speedup vs baseline: 4.5327x; 1.1311x over previous
import functools
import math

import jax
import jax.numpy as jnp
import numpy as np
from jax import lax
from jax.experimental import pallas as pl
from jax.experimental.pallas import tpu as pltpu
from jax.experimental.pallas import tpu_sc as plsc

_NUM_FEATS = 128
_TEMPERATURE = 10000.0
_NC = 2
_NS = 16
_NW = _NC * _NS
_CH = 128


def _sc_gather(table, labels):
    b = labels.shape[0]
    d = table.shape[1]
    per_w = b // _NW
    n_ch = per_w // _CH
    nbuf = 3
    mesh = plsc.VectorSubcoreMesh(core_axis_name="c", subcore_axis_name="s")

    @functools.partial(
        pl.kernel,
        mesh=mesh,
        out_type=jax.ShapeDtypeStruct((b, d), jnp.float32),
        scratch_types=[
            pltpu.VMEM((n_ch, _CH), jnp.int32),
            pltpu.VMEM((_CH, d), jnp.float32),
            pltpu.VMEM((_CH, d), jnp.float32),
            pltpu.VMEM((_CH, d), jnp.float32),
            pltpu.SemaphoreType.DMA,
            pltpu.SemaphoreType.DMA,
            pltpu.SemaphoreType.DMA,
            pltpu.SemaphoreType.DMA,
            pltpu.SemaphoreType.DMA,
            pltpu.SemaphoreType.DMA,
            pltpu.SemaphoreType.DMA,
        ],
    )
    def gather_kernel(lab_hbm, table_hbm, out_hbm, idx_v, r0, r1, r2,
                      isem, g0, g1, g2, s0, s1, s2):
        wid = lax.axis_index("s") * _NC + lax.axis_index("c")
        base = wid * per_w
        bufs = (r0, r1, r2)
        gsems = (g0, g1, g2)
        ssems = (s0, s1, s2)
        idx_copies = [
            pltpu.async_copy(lab_hbm.at[pl.ds(base + c * _CH, _CH)], idx_v.at[c], isem)
            for c in range(n_ch)
        ]
        for cp in idx_copies:
            cp.wait()
        gathers = {}
        stores = {}
        for c in range(min(nbuf, n_ch)):
            gathers[c] = pltpu.async_copy(
                table_hbm.at[idx_v.at[c]], bufs[c % nbuf], gsems[c % nbuf]
            )
        for c in range(n_ch):
            gathers[c].wait()
            stores[c] = pltpu.async_copy(
                bufs[c % nbuf], out_hbm.at[pl.ds(base + c * _CH, _CH)], ssems[c % nbuf]
            )
            nxt = c + nbuf
            if nxt < n_ch:
                stores[c].wait()
                gathers[nxt] = pltpu.async_copy(
                    table_hbm.at[idx_v.at[nxt]], bufs[nxt % nbuf], gsems[nxt % nbuf]
                )
        for c in range(max(0, n_ch - nbuf), n_ch):
            stores[c].wait()

    return gather_kernel(labels, table)


_TWO_OVER_PI = float(np.float32(2.0 / np.pi))
_MAGIC = 12582912.0
_P1 = float(np.float32(np.pi / 2))
_P2 = float(np.float32(np.pi / 2 - np.float64(np.float32(np.pi / 2))))
_P3 = float(
    np.float32(np.pi / 2 - np.float64(np.float32(np.pi / 2)) - np.float64(_P2))
)


def _sincos_sel(t, parity):
    j = (t * _TWO_OVER_PI + _MAGIC) - _MAGIC
    r = t - j * _P1
    r = r - j * _P2
    q = j.astype(jnp.int32) + parity
    r2 = r * r
    s = r + r * r2 * (-1.6658333e-1 + r2 * 8.0187609e-3)
    c = 1.0 + r2 * (-4.9985713e-1 + r2 * 4.0905953e-2)
    v = jnp.where((q & 1) == 0, s, c)
    sign = (q & 2) << 30
    return lax.bitcast_convert_type(
        lax.bitcast_convert_type(v, jnp.int32) ^ sign, jnp.float32
    )


def _tc_combine(gathered, coord_t, w, b, out_rows, blk_off, prev=None):
    n, d = gathered.shape
    nb = 1024

    def body(w_ref, b_ref, ct_ref, g_ref, *refs):
        out_ref = refs[-1]
        t = (
            lax.dot_general(
                ct_ref[...], w_ref[...], (((0,), (0,)), ((), ())),
                preferred_element_type=jnp.float32,
            )
            + b_ref[...]
        )
        ii = lax.broadcasted_iota(jnp.int32, (nb, 2 * _NUM_FEATS), 1)
        pos = _sincos_sel(t, ii & 1)
        out_ref[:, 0:256] = jnp.zeros((nb, 256), jnp.float32)
        out_ref[:, 256:512] = g_ref[...] + pos

    in_specs = [
        pl.BlockSpec((3, 2 * _NUM_FEATS), lambda i: (0, 0)),
        pl.BlockSpec((1, 2 * _NUM_FEATS), lambda i: (0, 0)),
        pl.BlockSpec((3, nb), lambda i: (0, i)),
        pl.BlockSpec((nb, d), lambda i: (i, 0)),
    ]
    inputs = [w, b, coord_t, gathered]
    io_alias = {}
    if prev is not None:
        in_specs.append(pl.BlockSpec(memory_space=pl.ANY))
        inputs.append(prev)
        io_alias = {4: 0}
    return pl.pallas_call(
        body,
        grid=(n // nb,),
        in_specs=in_specs,
        out_specs=pl.BlockSpec((nb, 2 * d), lambda i: (i + blk_off, 0)),
        out_shape=jax.ShapeDtypeStruct((out_rows, 2 * d), jnp.float32),
        input_output_aliases=io_alias,
    )(*inputs)


def kernel(point_coord, labels, pc_range, label_weight):
    n = labels.shape[0]
    half = n // 2
    g0 = _sc_gather(label_weight, labels[:half])
    g1 = _sc_gather(label_weight, labels[half:])
    i = np.arange(_NUM_FEATS, dtype=np.float32)
    dim_t = (_TEMPERATURE ** (2.0 * np.floor(i / 2.0) / _NUM_FEATS)).astype(np.float32)
    invd = jnp.asarray(1.0 / dim_t)
    scale = 2.0 * math.pi
    ax = scale / (pc_range[3] - pc_range[0])
    ay = scale / (pc_range[4] - pc_range[1])
    zero = jnp.zeros((_NUM_FEATS,), jnp.float32)
    w = jnp.stack(
        [
            jnp.concatenate([ax * invd, zero]),
            jnp.concatenate([zero, ay * invd]),
            jnp.zeros((2 * _NUM_FEATS,), jnp.float32),
        ]
    )
    b = jnp.concatenate([(-pc_range[0] * ax) * invd, (-pc_range[1] * ay) * invd])[None, :]
    coord_t = point_coord[0].T
    out0 = _tc_combine(g0, coord_t[:, :half], w, b, n, 0)
    out = _tc_combine(g1, coord_t[:, half:], w, b, n, half // 1024, prev=out0)
    return out[None]


# speedup vs baseline: 4.7759x; 1.0537x over previous
import functools
import math

import jax
import jax.numpy as jnp
import numpy as np
from jax import lax
from jax.experimental import pallas as pl
from jax.experimental.pallas import tpu as pltpu
from jax.experimental.pallas import tpu_sc as plsc

_NUM_FEATS = 128
_TEMPERATURE = 10000.0
_NC = 2
_NS = 16
_NW = _NC * _NS
_CH = 128


def _sc_gather(table, labels):
    b = labels.shape[0]
    d = table.shape[1]
    per_w = b // _NW
    n_ch = per_w // _CH
    nbuf = 3
    mesh = plsc.VectorSubcoreMesh(core_axis_name="c", subcore_axis_name="s")

    @functools.partial(
        pl.kernel,
        mesh=mesh,
        out_type=jax.ShapeDtypeStruct((b, d), jnp.float32),
        scratch_types=[
            pltpu.VMEM((n_ch, _CH), jnp.int32),
            pltpu.VMEM((_CH, d), jnp.float32),
            pltpu.VMEM((_CH, d), jnp.float32),
            pltpu.VMEM((_CH, d), jnp.float32),
            pltpu.SemaphoreType.DMA,
            pltpu.SemaphoreType.DMA,
            pltpu.SemaphoreType.DMA,
            pltpu.SemaphoreType.DMA,
            pltpu.SemaphoreType.DMA,
            pltpu.SemaphoreType.DMA,
            pltpu.SemaphoreType.DMA,
        ],
    )
    def gather_kernel(lab_hbm, table_hbm, out_hbm, idx_v, r0, r1, r2,
                      isem, g0, g1, g2, s0, s1, s2):
        wid = lax.axis_index("s") * _NC + lax.axis_index("c")
        base = wid * per_w
        bufs = (r0, r1, r2)
        gsems = (g0, g1, g2)
        ssems = (s0, s1, s2)
        idx_copies = [
            pltpu.async_copy(lab_hbm.at[pl.ds(base + c * _CH, _CH)], idx_v.at[c], isem)
            for c in range(n_ch)
        ]
        for cp in idx_copies:
            cp.wait()
        gathers = {}
        stores = {}
        for c in range(min(nbuf, n_ch)):
            gathers[c] = pltpu.async_copy(
                table_hbm.at[idx_v.at[c]], bufs[c % nbuf], gsems[c % nbuf]
            )
        for c in range(n_ch):
            gathers[c].wait()
            stores[c] = pltpu.async_copy(
                bufs[c % nbuf], out_hbm.at[pl.ds(base + c * _CH, _CH)], ssems[c % nbuf]
            )
            nxt = c + nbuf
            if nxt < n_ch:
                stores[c].wait()
                gathers[nxt] = pltpu.async_copy(
                    table_hbm.at[idx_v.at[nxt]], bufs[nxt % nbuf], gsems[nxt % nbuf]
                )
        for c in range(max(0, n_ch - nbuf), n_ch):
            stores[c].wait()

    return gather_kernel(labels, table)


_TWO_OVER_PI = float(np.float32(2.0 / np.pi))
_MAGIC = 12582912.0
_P1 = float(np.float32(np.pi / 2))
_P2 = float(np.float32(np.pi / 2 - np.float64(np.float32(np.pi / 2))))
_P3 = float(
    np.float32(np.pi / 2 - np.float64(np.float32(np.pi / 2)) - np.float64(_P2))
)


def _sincos_sel(t, parity):
    j = (t * _TWO_OVER_PI + _MAGIC) - _MAGIC
    r = t - j * _P1
    r = r - j * _P2
    q = j.astype(jnp.int32) + parity
    r2 = r * r
    s = r + r * r2 * (-1.6658333e-1 + r2 * 8.0187609e-3)
    c = 1.0 + r2 * (-4.9985713e-1 + r2 * 4.0905953e-2)
    v = jnp.where((q & 1) == 0, s, c)
    sign = (q & 2) << 30
    return lax.bitcast_convert_type(
        lax.bitcast_convert_type(v, jnp.int32) ^ sign, jnp.float32
    )


def _tc_combine(gathered, coord_t, w, b, out_rows, blk_off, prev=None):
    n, d = gathered.shape
    nb = 2048

    def body(w_ref, b_ref, ct_ref, g_ref, *refs):
        out_ref = refs[-1]
        t = (
            lax.dot_general(
                ct_ref[...], w_ref[...], (((0,), (0,)), ((), ())),
                preferred_element_type=jnp.float32,
            )
            + b_ref[...]
        )
        ii = lax.broadcasted_iota(jnp.int32, (nb, 2 * _NUM_FEATS), 1)
        pos = _sincos_sel(t, ii & 1)
        out_ref[:, 0:256] = jnp.zeros((nb, 256), jnp.float32)
        out_ref[:, 256:512] = g_ref[...] + pos

    in_specs = [
        pl.BlockSpec((3, 2 * _NUM_FEATS), lambda i: (0, 0)),
        pl.BlockSpec((1, 2 * _NUM_FEATS), lambda i: (0, 0)),
        pl.BlockSpec((3, nb), lambda i: (0, i)),
        pl.BlockSpec((nb, d), lambda i: (i, 0)),
    ]
    inputs = [w, b, coord_t, gathered]
    io_alias = {}
    if prev is not None:
        in_specs.append(pl.BlockSpec(memory_space=pl.ANY))
        inputs.append(prev)
        io_alias = {4: 0}
    return pl.pallas_call(
        body,
        grid=(n // nb,),
        in_specs=in_specs,
        out_specs=pl.BlockSpec((nb, 2 * d), lambda i: (i + blk_off, 0)),
        out_shape=jax.ShapeDtypeStruct((out_rows, 2 * d), jnp.float32),
        input_output_aliases=io_alias,
    )(*inputs)


def kernel(point_coord, labels, pc_range, label_weight):
    n = labels.shape[0]
    half = n // 2
    g0 = _sc_gather(label_weight, labels[:half])
    g1 = _sc_gather(label_weight, labels[half:])
    i = np.arange(_NUM_FEATS, dtype=np.float32)
    dim_t = (_TEMPERATURE ** (2.0 * np.floor(i / 2.0) / _NUM_FEATS)).astype(np.float32)
    invd = jnp.asarray(1.0 / dim_t)
    scale = 2.0 * math.pi
    ax = scale / (pc_range[3] - pc_range[0])
    ay = scale / (pc_range[4] - pc_range[1])
    zero = jnp.zeros((_NUM_FEATS,), jnp.float32)
    w = jnp.stack(
        [
            jnp.concatenate([ax * invd, zero]),
            jnp.concatenate([zero, ay * invd]),
            jnp.zeros((2 * _NUM_FEATS,), jnp.float32),
        ]
    )
    b = jnp.concatenate([(-pc_range[0] * ax) * invd, (-pc_range[1] * ay) * invd])[None, :]
    coord_t = point_coord[0].T
    out0 = _tc_combine(g0, coord_t[:, :half], w, b, n, 0)
    out = _tc_combine(g1, coord_t[:, half:], w, b, n, half // 2048, prev=out0)
    return out[None]


# speedup vs baseline: 4.8500x; 1.0155x over previous
import functools
import math

import jax
import jax.numpy as jnp
import numpy as np
from jax import lax
from jax.experimental import pallas as pl
from jax.experimental.pallas import tpu as pltpu
from jax.experimental.pallas import tpu_sc as plsc

_NUM_FEATS = 128
_TEMPERATURE = 10000.0
_NC = 2
_NS = 16
_NW = _NC * _NS
_CH = 128


def _sc_gather(table, labels):
    b = labels.shape[0]
    d = table.shape[1]
    per_w = b // _NW
    n_ch = per_w // _CH
    nbuf = 3
    mesh = plsc.VectorSubcoreMesh(core_axis_name="c", subcore_axis_name="s")

    @functools.partial(
        pl.kernel,
        mesh=mesh,
        out_type=jax.ShapeDtypeStruct((b, d), jnp.float32),
        scratch_types=[
            pltpu.VMEM((n_ch, _CH), jnp.int32),
            pltpu.VMEM((_CH, d), jnp.float32),
            pltpu.VMEM((_CH, d), jnp.float32),
            pltpu.VMEM((_CH, d), jnp.float32),
            pltpu.SemaphoreType.DMA,
            pltpu.SemaphoreType.DMA,
            pltpu.SemaphoreType.DMA,
            pltpu.SemaphoreType.DMA,
            pltpu.SemaphoreType.DMA,
            pltpu.SemaphoreType.DMA,
            pltpu.SemaphoreType.DMA,
        ],
    )
    def gather_kernel(lab_hbm, table_hbm, out_hbm, idx_v, r0, r1, r2,
                      isem, g0, g1, g2, s0, s1, s2):
        wid = lax.axis_index("s") * _NC + lax.axis_index("c")
        base = wid * per_w
        bufs = (r0, r1, r2)
        gsems = (g0, g1, g2)
        ssems = (s0, s1, s2)
        idx_copies = [
            pltpu.async_copy(lab_hbm.at[pl.ds(base + c * _CH, _CH)], idx_v.at[c], isem)
            for c in range(n_ch)
        ]
        for cp in idx_copies:
            cp.wait()
        gathers = {}
        stores = {}
        for c in range(min(nbuf, n_ch)):
            gathers[c] = pltpu.async_copy(
                table_hbm.at[idx_v.at[c]], bufs[c % nbuf], gsems[c % nbuf]
            )
        for c in range(n_ch):
            gathers[c].wait()
            stores[c] = pltpu.async_copy(
                bufs[c % nbuf], out_hbm.at[pl.ds(base + c * _CH, _CH)], ssems[c % nbuf]
            )
            nxt = c + nbuf
            if nxt < n_ch:
                stores[c].wait()
                gathers[nxt] = pltpu.async_copy(
                    table_hbm.at[idx_v.at[nxt]], bufs[nxt % nbuf], gsems[nxt % nbuf]
                )
        for c in range(max(0, n_ch - nbuf), n_ch):
            stores[c].wait()

    return gather_kernel(labels, table)


_TWO_OVER_PI = float(np.float32(2.0 / np.pi))
_MAGIC = 12582912.0
_P1 = float(np.float32(np.pi / 2))
_P2 = float(np.float32(np.pi / 2 - np.float64(np.float32(np.pi / 2))))
_P3 = float(
    np.float32(np.pi / 2 - np.float64(np.float32(np.pi / 2)) - np.float64(_P2))
)


def _sincos_sel(t, parity):
    j = (t * _TWO_OVER_PI + _MAGIC) - _MAGIC
    r = t - j * _P1
    r = r - j * _P2
    q = j.astype(jnp.int32) + parity
    r2 = r * r
    s = r + r * r2 * (-1.6658333e-1 + r2 * 8.0187609e-3)
    c = 1.0 + r2 * (-4.9985713e-1 + r2 * 4.0905953e-2)
    v = jnp.where((q & 1) == 0, s, c)
    sign = (q & 2) << 30
    return lax.bitcast_convert_type(
        lax.bitcast_convert_type(v, jnp.int32) ^ sign, jnp.float32
    )


def _tc_combine(gathered, coord_t, w, b, out_rows, blk_off, prev=None):
    n, d = gathered.shape
    nb = 4096

    def body(w_ref, b_ref, ct_ref, g_ref, *refs):
        out_ref = refs[-1]
        t = (
            lax.dot_general(
                ct_ref[...], w_ref[...], (((0,), (0,)), ((), ())),
                preferred_element_type=jnp.float32,
            )
            + b_ref[...]
        )
        ii = lax.broadcasted_iota(jnp.int32, (nb, 2 * _NUM_FEATS), 1)
        pos = _sincos_sel(t, ii & 1)
        out_ref[:, 0:256] = jnp.zeros((nb, 256), jnp.float32)
        out_ref[:, 256:512] = g_ref[...] + pos

    in_specs = [
        pl.BlockSpec((3, 2 * _NUM_FEATS), lambda i: (0, 0)),
        pl.BlockSpec((1, 2 * _NUM_FEATS), lambda i: (0, 0)),
        pl.BlockSpec((3, nb), lambda i: (0, i)),
        pl.BlockSpec((nb, d), lambda i: (i, 0)),
    ]
    inputs = [w, b, coord_t, gathered]
    io_alias = {}
    if prev is not None:
        in_specs.append(pl.BlockSpec(memory_space=pl.ANY))
        inputs.append(prev)
        io_alias = {4: 0}
    return pl.pallas_call(
        body,
        grid=(n // nb,),
        in_specs=in_specs,
        out_specs=pl.BlockSpec((nb, 2 * d), lambda i: (i + blk_off, 0)),
        out_shape=jax.ShapeDtypeStruct((out_rows, 2 * d), jnp.float32),
        input_output_aliases=io_alias,
    )(*inputs)


def kernel(point_coord, labels, pc_range, label_weight):
    n = labels.shape[0]
    half = n // 2
    g0 = _sc_gather(label_weight, labels[:half])
    g1 = _sc_gather(label_weight, labels[half:])
    i = np.arange(_NUM_FEATS, dtype=np.float32)
    dim_t = (_TEMPERATURE ** (2.0 * np.floor(i / 2.0) / _NUM_FEATS)).astype(np.float32)
    invd = jnp.asarray(1.0 / dim_t)
    scale = 2.0 * math.pi
    ax = scale / (pc_range[3] - pc_range[0])
    ay = scale / (pc_range[4] - pc_range[1])
    zero = jnp.zeros((_NUM_FEATS,), jnp.float32)
    w = jnp.stack(
        [
            jnp.concatenate([ax * invd, zero]),
            jnp.concatenate([zero, ay * invd]),
            jnp.zeros((2 * _NUM_FEATS,), jnp.float32),
        ]
    )
    b = jnp.concatenate([(-pc_range[0] * ax) * invd, (-pc_range[1] * ay) * invd])[None, :]
    coord_t = point_coord[0].T
    out0 = _tc_combine(g0, coord_t[:, :half], w, b, n, 0)
    out = _tc_combine(g1, coord_t[:, half:], w, b, n, half // 4096, prev=out0)
    return out[None]


# speedup vs baseline: 4.8536x; 1.0007x over previous
import functools
import math

import jax
import jax.numpy as jnp
import numpy as np
from jax import lax
from jax.experimental import pallas as pl
from jax.experimental.pallas import tpu as pltpu
from jax.experimental.pallas import tpu_sc as plsc

_NUM_FEATS = 128
_TEMPERATURE = 10000.0
_NC = 2
_NS = 16
_NW = _NC * _NS
_CH = 128


def _sc_gather(table, labels):
    b = labels.shape[0]
    d = table.shape[1]
    per_w = b // _NW
    n_ch = per_w // _CH
    nbuf = 3
    mesh = plsc.VectorSubcoreMesh(core_axis_name="c", subcore_axis_name="s")

    @functools.partial(
        pl.kernel,
        mesh=mesh,
        out_type=jax.ShapeDtypeStruct((b, d), jnp.float32),
        scratch_types=[
            pltpu.VMEM((n_ch, _CH), jnp.int32),
            pltpu.VMEM((_CH, d), jnp.float32),
            pltpu.VMEM((_CH, d), jnp.float32),
            pltpu.VMEM((_CH, d), jnp.float32),
            pltpu.SemaphoreType.DMA,
            pltpu.SemaphoreType.DMA,
            pltpu.SemaphoreType.DMA,
            pltpu.SemaphoreType.DMA,
            pltpu.SemaphoreType.DMA,
            pltpu.SemaphoreType.DMA,
            pltpu.SemaphoreType.DMA,
        ],
    )
    def gather_kernel(lab_hbm, table_hbm, out_hbm, idx_v, r0, r1, r2,
                      isem, g0, g1, g2, s0, s1, s2):
        wid = lax.axis_index("s") * _NC + lax.axis_index("c")
        base = wid * per_w
        bufs = (r0, r1, r2)
        gsems = (g0, g1, g2)
        ssems = (s0, s1, s2)
        idx_copies = [
            pltpu.async_copy(lab_hbm.at[pl.ds(base + c * _CH, _CH)], idx_v.at[c], isem)
            for c in range(n_ch)
        ]
        for cp in idx_copies:
            cp.wait()
        gathers = {}
        stores = {}
        for c in range(min(nbuf, n_ch)):
            gathers[c] = pltpu.async_copy(
                table_hbm.at[idx_v.at[c]], bufs[c % nbuf], gsems[c % nbuf]
            )
        for c in range(n_ch):
            gathers[c].wait()
            stores[c] = pltpu.async_copy(
                bufs[c % nbuf], out_hbm.at[pl.ds(base + c * _CH, _CH)], ssems[c % nbuf]
            )
            nxt = c + nbuf
            if nxt < n_ch:
                stores[c].wait()
                gathers[nxt] = pltpu.async_copy(
                    table_hbm.at[idx_v.at[nxt]], bufs[nxt % nbuf], gsems[nxt % nbuf]
                )
        for c in range(max(0, n_ch - nbuf), n_ch):
            stores[c].wait()

    return gather_kernel(labels, table)


_TWO_OVER_PI = float(np.float32(2.0 / np.pi))
_MAGIC = 12582912.0
_P1 = float(np.float32(np.pi / 2))
_P2 = float(np.float32(np.pi / 2 - np.float64(np.float32(np.pi / 2))))


def _sincos_sel(t, parity):
    j = (t * _TWO_OVER_PI + _MAGIC) - _MAGIC
    r = t - j * _P1
    r = r - j * _P2
    q = j.astype(jnp.int32) + parity
    r2 = r * r
    s = r + r * r2 * (-1.6658333e-1 + r2 * 8.0187609e-3)
    c = 1.0 + r2 * (-4.9985713e-1 + r2 * 4.0905953e-2)
    v = jnp.where((q & 1) == 0, s, c)
    sign = (q & 2) << 30
    return lax.bitcast_convert_type(
        lax.bitcast_convert_type(v, jnp.int32) ^ sign, jnp.float32
    )


def _tc_combine(gathered, coord_t, w, b, out_rows, blk_off, prev=None):
    n, d = gathered.shape
    nb = 4096

    def body(w_ref, b_ref, ct_ref, g_ref, *refs):
        out_ref = refs[-1]
        t = (
            lax.dot_general(
                ct_ref[...], w_ref[...], (((0,), (0,)), ((), ())),
                preferred_element_type=jnp.float32,
            )
            + b_ref[...]
        )
        ii = lax.broadcasted_iota(jnp.int32, (nb, 2 * _NUM_FEATS), 1)
        pos = _sincos_sel(t, ii & 1)
        out_ref[0, :, 0:256] = jnp.zeros((nb, 256), jnp.float32)
        out_ref[0, :, 256:512] = g_ref[...] + pos

    in_specs = [
        pl.BlockSpec((3, 2 * _NUM_FEATS), lambda i: (0, 0)),
        pl.BlockSpec((1, 2 * _NUM_FEATS), lambda i: (0, 0)),
        pl.BlockSpec((3, nb), lambda i: (0, i)),
        pl.BlockSpec((nb, d), lambda i: (i, 0)),
    ]
    inputs = [w, b, coord_t, gathered]
    io_alias = {}
    if prev is not None:
        in_specs.append(pl.BlockSpec(memory_space=pl.ANY))
        inputs.append(prev)
        io_alias = {4: 0}
    return pl.pallas_call(
        body,
        grid=(n // nb,),
        in_specs=in_specs,
        out_specs=pl.BlockSpec((1, nb, 2 * d), lambda i: (0, i + blk_off, 0)),
        out_shape=jax.ShapeDtypeStruct((1, out_rows, 2 * d), jnp.float32),
        input_output_aliases=io_alias,
    )(*inputs)


def kernel(point_coord, labels, pc_range, label_weight):
    n = labels.shape[0]
    half = n // 2
    g0 = _sc_gather(label_weight, labels[:half])
    g1 = _sc_gather(label_weight, labels[half:])
    i = np.arange(_NUM_FEATS, dtype=np.float32)
    dim_t = (_TEMPERATURE ** (2.0 * np.floor(i / 2.0) / _NUM_FEATS)).astype(np.float32)
    invd = jnp.asarray(1.0 / dim_t)
    scale = 2.0 * math.pi
    ax = scale / (pc_range[3] - pc_range[0])
    ay = scale / (pc_range[4] - pc_range[1])
    zero = jnp.zeros((_NUM_FEATS,), jnp.float32)
    w = jnp.stack(
        [
            jnp.concatenate([ax * invd, zero]),
            jnp.concatenate([zero, ay * invd]),
            jnp.zeros((2 * _NUM_FEATS,), jnp.float32),
        ]
    )
    b = jnp.concatenate([(-pc_range[0] * ax) * invd, (-pc_range[1] * ay) * invd])[None, :]
    coord_t = point_coord[0].T
    out0 = _tc_combine(g0, coord_t[:, :half], w, b, n, 0)
    out = _tc_combine(g1, coord_t[:, half:], w, b, n, half // 4096, prev=out0)
    return out
